# baseline (speedup 1.0000x reference)
import functools

import jax
import jax.numpy as jnp
from jax import lax
from jax.experimental import pallas as pl
from jax.experimental.pallas import tpu as pltpu
from jax.experimental.pallas import tpu_sc as plsc

_NC = 2
_NS = 16
_NW = _NC * _NS
_DEG_BLK = 1024
_EP_HO = 851968
_EP_BIP = 1605632
_NPAD = 50176
_ROWS_PER_TILE = _NPAD // _NS
_ZROWS = 64


def _mesh():
    return plsc.VectorSubcoreMesh(
        core_axis_name="c", subcore_axis_name="s",
        num_cores=_NC, num_subcores=_NS)


def _pad_edges(src, dst, ew, ep, pad_dst_base):
    e = src.shape[0]
    npad = ep - e
    pidx = jnp.arange(npad, dtype=jnp.int32)
    srcp = jnp.concatenate([src, pidx % 4096])
    dstp = jnp.concatenate([dst, pad_dst_base + (pidx % 128)])
    out = [srcp.reshape(ep // 128, 128), dstp.reshape(ep // 128, 128)]
    if ew is not None:
        ewp = jnp.concatenate([ew, jnp.zeros((npad,), jnp.float32)])
        out.append(ewp.reshape(ep // 128, 128))
    return out


def _zero_acc2d(zbuf, acc, tile, ncols):
    @pl.loop(0, _ZROWS)
    def _fill(i):
        for cc in range(ncols // 16):
            zbuf[i, pl.ds(cc * 16, 16)] = jnp.zeros((16,), jnp.float32)

    @pl.loop(0, _ROWS_PER_TILE // _ZROWS)
    def _z(k):
        pltpu.sync_copy(zbuf, acc.at[pl.ds(tile * _ROWS_PER_TILE + k * _ZROWS, _ZROWS)])


def _sc_deg(dst2d, ew2d, nb):
    nj = _DEG_BLK // 128

    @functools.partial(
        pl.kernel,
        out_type=jax.ShapeDtypeStruct((_NC, _NPAD), jnp.float32),
        mesh=_mesh(),
        compiler_params=pltpu.CompilerParams(use_tc_tiling_on_sc=False),
        scratch_types=[
            pltpu.VMEM_SHARED((_NPAD,), jnp.float32),
            pltpu.VMEM((nj, 128), jnp.int32), pltpu.VMEM((nj, 128), jnp.int32),
            pltpu.VMEM((nj, 128), jnp.float32), pltpu.VMEM((nj, 128), jnp.float32),
            pltpu.VMEM((_NPAD // _NS,), jnp.float32),
            pltpu.SemaphoreType.DMA, pltpu.SemaphoreType.DMA,
        ],
    )
    def k(dst_hbm, ew_hbm, out_hbm, acc, dst0, dst1, ew0, ew1, zbuf,
          isem0, isem1):
        dstb, ewb, isem = (dst0, dst1), (ew0, ew1), (isem0, isem1)
        c = lax.axis_index("c")
        s = lax.axis_index("s")
        w = s * _NC + c
        npt = _NPAD // _NS
        base = w * nb

        @pl.loop(0, npt // 16)
        def _fill(i):
            zbuf[pl.ds(i * 16, 16)] = jnp.zeros((16,), jnp.float32)

        pltpu.sync_copy(zbuf, acc.at[pl.ds(s * npt, npt)])

        plsc.subcore_barrier()

        def stage(b, k_):
            row0 = (base + b) * nj
            pltpu.async_copy(dst_hbm.at[pl.ds(row0, nj)], dstb[k_], isem[k_])
            pltpu.async_copy(ew_hbm.at[pl.ds(row0, nj)], ewb[k_], isem[k_])

        def wait_stage(k_):
            pltpu.make_async_copy(dst_hbm.at[pl.ds(0, nj)], dstb[k_], isem[k_]).wait()
            pltpu.make_async_copy(ew_hbm.at[pl.ds(0, nj)], ewb[k_], isem[k_]).wait()

        stage(0, 0)
        stage(1, 1)

        @pl.loop(0, nb // 2)
        def _blk(t):
            for k_ in (0, 1):
                b = 2 * t + k_
                wait_stage(k_)
                for j in range(nj):
                    pltpu.sync_copy(ewb[k_].at[j], acc.at[dstb[k_].at[j]], add=True)
                stage(jnp.minimum(b + 2, nb - 1), k_)

        wait_stage(0)
        wait_stage(1)

        plsc.subcore_barrier()
        pltpu.sync_copy(acc.at[pl.ds(s * npt, npt)], out_hbm.at[c, pl.ds(s * npt, npt)])

    return k(dst2d, ew2d)


def _sc_edge_pass(h, src2d, dst2d, ew2d, nb, blk, ncols):
    nj = blk // 128
    weighted = ew2d is not None
    scratch = [
        pltpu.VMEM_SHARED((_NPAD, ncols), jnp.float32),
        pltpu.VMEM((nj, 128), jnp.int32), pltpu.VMEM((nj, 128), jnp.int32),
        pltpu.VMEM((nj, 128), jnp.int32), pltpu.VMEM((nj, 128), jnp.int32),
        pltpu.VMEM((nj, 128), jnp.int32), pltpu.VMEM((nj, 128), jnp.int32),
        pltpu.VMEM((blk, ncols), jnp.float32),
        pltpu.VMEM((blk, ncols), jnp.float32),
        pltpu.VMEM((_ZROWS, ncols), jnp.float32),
        pltpu.SemaphoreType.DMA, pltpu.SemaphoreType.DMA,
        pltpu.SemaphoreType.DMA, pltpu.SemaphoreType.DMA,
        pltpu.SemaphoreType.DMA, pltpu.SemaphoreType.DMA,
    ]
    if weighted:
        scratch += [pltpu.VMEM((nj, 128), jnp.float32),
                    pltpu.VMEM((nj, 128), jnp.float32)]

    def body(*args):
        if weighted:
            (h_hbm, src_hbm, dst_hbm, ew_hbm, out_hbm, acc,
             src0, src1, dst0, dst1, dsts0, dsts1, rows0, rows1, zbuf,
             gsem0, gsem1, isem0, isem1, ssem0, ssem1, ew0, ew1) = args
            ewb = (ew0, ew1)
        else:
            (h_hbm, src_hbm, dst_hbm, out_hbm, acc,
             src0, src1, dst0, dst1, dsts0, dsts1, rows0, rows1, zbuf,
             gsem0, gsem1, isem0, isem1, ssem0, ssem1) = args
            ewb = (None, None)
        srcb, dstb, rowsb = (src0, src1), (dst0, dst1), (rows0, rows1)
        dstsb = (dsts0, dsts1)
        gsem, isem, ssem = (gsem0, gsem1), (isem0, isem1), (ssem0, ssem1)
        c = lax.axis_index("c")
        s = lax.axis_index("s")
        w = s * _NC + c
        base = w * nb

        _zero_acc2d(zbuf, acc, s, ncols)
        plsc.subcore_barrier()

        def stage(b, k, sem):
            row0 = (base + b) * nj
            if sem is None:
                pltpu.sync_copy(src_hbm.at[pl.ds(row0, nj)], srcb[k])
                pltpu.sync_copy(dst_hbm.at[pl.ds(row0, nj)], dstb[k])
                if weighted:
                    pltpu.sync_copy(ew_hbm.at[pl.ds(row0, nj)], ewb[k])
            else:
                pltpu.async_copy(src_hbm.at[pl.ds(row0, nj)], srcb[k], sem)
                pltpu.async_copy(dst_hbm.at[pl.ds(row0, nj)], dstb[k], sem)
                if weighted:
                    pltpu.async_copy(ew_hbm.at[pl.ds(row0, nj)], ewb[k], sem)

        def wait_stage(k):
            pltpu.make_async_copy(src_hbm.at[pl.ds(0, nj)], srcb[k], isem[k]).wait()
            pltpu.make_async_copy(dst_hbm.at[pl.ds(0, nj)], dstb[k], isem[k]).wait()
            if weighted:
                pltpu.make_async_copy(ew_hbm.at[pl.ds(0, nj)], ewb[k], isem[k]).wait()

        def start_gathers(k):
            for j in range(nj):
                pltpu.async_copy(h_hbm.at[srcb[k].at[j]],
                                 rowsb[k].at[pl.ds(j * 128, 128)], gsem[k])

        def wait_gathers(k):
            for j in range(nj):
                pltpu.make_async_copy(
                    h_hbm.at[srcb[k].at[j]],
                    rowsb[k].at[pl.ds(j * 128, 128)], gsem[k]).wait()

        def scale(k):
            rows, ew_v = rowsb[k], ewb[k]

            @pl.loop(0, blk // 16, unroll=2)
            def _grp(g):
                r = g // 8
                col = (g % 8) * 16
                ewv = ew_v[r, pl.ds(col, 16)]
                for j in range(16):
                    e = g * 16 + j
                    sc = ewv[j]
                    for cc in range(ncols // 16):
                        rows[e, pl.ds(cc * 16, 16)] = (
                            rows[e, pl.ds(cc * 16, 16)] * sc)

        def scatter(k):
            for j in range(nj):
                pltpu.sync_copy(rowsb[k].at[pl.ds(j * 128, 128)],
                                acc.at[dstb[k].at[j]], add=True)

        stage(0, 0, None)
        start_gathers(0)
        stage(1, 1, isem[1])

        @pl.loop(0, nb // 2)
        def _outer(t):
            for k in (0, 1):
                b = 2 * t + k
                wait_gathers(k)
                wait_stage(k ^ 1)
                start_gathers(k ^ 1)
                if weighted:
                    scale(k)
                scatter(k)
                stage(jnp.minimum(b + 2, nb - 1), k, isem[k])

        wait_gathers(0)
        wait_stage(1)

        plsc.subcore_barrier()
        pltpu.sync_copy(acc.at[pl.ds(s * _ROWS_PER_TILE, _ROWS_PER_TILE)],
                        out_hbm.at[c, pl.ds(s * _ROWS_PER_TILE, _ROWS_PER_TILE)])

    kern = functools.partial(
        pl.kernel,
        out_type=jax.ShapeDtypeStruct((_NC, _NPAD, ncols), jnp.float32),
        mesh=_mesh(),
        compiler_params=pltpu.CompilerParams(use_tc_tiling_on_sc=False),
        scratch_types=scratch,
    )(body)
    if weighted:
        return kern(h, src2d, dst2d, ew2d)
    return kern(h, src2d, dst2d)


def _elu(a):
    return jnp.where(a > 0, a, jnp.exp(jnp.minimum(a, 0.0)) - 1.0)


_TCBLK = 8192


def _tc_call(body, n, outd, *args):
    grid = ((n + _TCBLK - 1) // _TCBLK,)
    arrs, specs = zip(*args)
    return pl.pallas_call(
        body,
        grid=grid,
        in_specs=list(specs),
        out_specs=pl.BlockSpec((_TCBLK, outd), lambda i: (i, 0)),
        out_shape=jax.ShapeDtypeStruct((n, outd), jnp.float32),
    )(*arrs)


def _row_spec(d):
    return pl.BlockSpec((_TCBLK, d), lambda i: (i, 0))


def _full_spec(s0, s1):
    return pl.BlockSpec((s0, s1), lambda i: (0, 0))


def _part_spec(d, which):
    return pl.BlockSpec((1, _TCBLK, d), lambda i, _w=which: (_w, i, 0))


def kernel(x_ho, edge_index, edge_weight, x_fo, edge_index_fo, edge_weight_fo,
           edge_index_hon_to_fon, num_ho_nodes, num_fo_nodes,
           W220, b220, W221, b221, W110, b110, W111, b111,
           Wl1, bl1, Wl2, bl2, Wmlp, bmlp):
    n_ho = x_ho.shape[0]
    n_fo = x_fo.shape[0]

    src2d, dst2d, ew2d = _pad_edges(
        edge_index[0], edge_index[1], edge_weight, _EP_HO, 0)
    bsrc2d, bdst2d = _pad_edges(
        edge_index_hon_to_fon[0], edge_index_hon_to_fon[1], None, _EP_BIP, n_ho)

    degp = _sc_deg(dst2d, ew2d, _EP_HO // (_NW * _DEG_BLK)).reshape(_NC, _NPAD, 1)

    def b_body(d0, d1, x_r, w_r, g_r, dis_r):
        deg = d0[0] + d1[0] + 1.0
        dis = lax.rsqrt(deg)
        h = jnp.dot(x_r[...], w_r[...], preferred_element_type=jnp.float32)
        g_r[...] = dis * h
        dis_r[...] = dis

    grid = ((n_ho + _TCBLK - 1) // _TCBLK,)
    g1, dis = pl.pallas_call(
        b_body,
        grid=grid,
        in_specs=[_part_spec(1, 0), _part_spec(1, 1),
                  _row_spec(x_ho.shape[1]), _full_spec(*W220.shape)],
        out_specs=[_row_spec(W220.shape[1]), _row_spec(1)],
        out_shape=[jax.ShapeDtypeStruct((n_ho, W220.shape[1]), jnp.float32),
                   jax.ShapeDtypeStruct((n_ho, 1), jnp.float32)],
    )(degp, degp, x_ho, W220)

    a1 = _sc_edge_pass(g1, src2d, dst2d, ew2d, _EP_HO // (_NW * 1024), 1024, W220.shape[1])

    def d_body(a0, a1_, g_r, dis_r, b_r, w_r, o_r):
        dis = dis_r[...]
        x = _elu(dis * (a0[0] + a1_[0] + g_r[...]) + b_r[...])
        o_r[...] = dis * jnp.dot(x, w_r[...], preferred_element_type=jnp.float32)

    g2 = _tc_call(d_body, n_ho, W221.shape[1],
                  (a1, _part_spec(W220.shape[1], 0)),
                  (a1, _part_spec(W220.shape[1], 1)),
                  (g1, _row_spec(W220.shape[1])),
                  (dis, _row_spec(1)),
                  (b220.reshape(1, -1), _full_spec(1, b220.shape[0])),
                  (W221, _full_spec(*W221.shape)))

    a2 = _sc_edge_pass(g2, src2d, dst2d, ew2d, _EP_HO // (_NW * 256), 256, W221.shape[1])

    def f_body(a0, a1_, g_r, dis_r, b_r, w_r, bl_r, o_r):
        dis = dis_r[...]
        x2 = _elu(dis * (a0[0] + a1_[0] + g_r[...]) + b_r[...])
        o_r[...] = jnp.dot(x2, w_r[...], preferred_element_type=jnp.float32) + bl_r[...]

    h_src = _tc_call(f_body, n_ho, Wl1.shape[1],
                     (a2, _part_spec(W221.shape[1], 0)),
                     (a2, _part_spec(W221.shape[1], 1)),
                     (g2, _row_spec(W221.shape[1])),
                     (dis, _row_spec(1)),
                     (b221.reshape(1, -1), _full_spec(1, b221.shape[0])),
                     (Wl1, _full_spec(*Wl1.shape)),
                     (bl1.reshape(1, -1), _full_spec(1, bl1.shape[0])))

    a3 = _sc_edge_pass(h_src, bsrc2d, bdst2d, None, _EP_BIP // (_NW * 256), 256, Wl1.shape[1])

    def h_body(a0, a1_, w_r, b_r, o_r):
        a = _elu(a0[0] + a1_[0])
        o_r[...] = jnp.dot(a, w_r[...], preferred_element_type=jnp.float32) + b_r[...]

    out_top = _tc_call(h_body, n_ho, Wmlp.shape[1],
                       (a3, _part_spec(Wl1.shape[1], 0)),
                       (a3, _part_spec(Wl1.shape[1], 1)),
                       (Wmlp, _full_spec(*Wmlp.shape)),
                       (bmlp.reshape(1, -1), _full_spec(1, bmlp.shape[0])))

    out_bot = jnp.broadcast_to(bmlp, (n_fo - n_ho, bmlp.shape[0]))
    return jnp.concatenate([out_top, out_bot], axis=0)


# speedup vs baseline: 1.1932x; 1.1932x over previous
import functools

import jax
import jax.numpy as jnp
from jax import lax
from jax.experimental import pallas as pl
from jax.experimental.pallas import tpu as pltpu
from jax.experimental.pallas import tpu_sc as plsc

_NC = 2
_NS = 16
_NW = _NC * _NS
_DEG_BLK = 1024
_EP_HO = 851968
_EP_BIP = 1605632
_NPAD = 50176
_ROWS_PER_TILE = _NPAD // _NS
_ZROWS = 64


def _mesh():
    return plsc.VectorSubcoreMesh(
        core_axis_name="c", subcore_axis_name="s",
        num_cores=_NC, num_subcores=_NS)


def _pad_edges(src, dst, ew, ep, pad_dst_base):
    e = src.shape[0]
    npad = ep - e
    pidx = jnp.arange(npad, dtype=jnp.int32)
    srcp = jnp.concatenate([src, pidx % 4096])
    dstp = jnp.concatenate([dst, pad_dst_base + (pidx % 128)])
    out = [srcp.reshape(ep // 128, 128), dstp.reshape(ep // 128, 128)]
    if ew is not None:
        ewp = jnp.concatenate([ew, jnp.zeros((npad,), jnp.float32)])
        out.append(ewp.reshape(ep // 128, 128))
    return out


def _zero_acc2d(zbuf, acc, tile, ncols):
    @pl.loop(0, _ZROWS)
    def _fill(i):
        for cc in range(ncols // 16):
            zbuf[i, pl.ds(cc * 16, 16)] = jnp.zeros((16,), jnp.float32)

    @pl.loop(0, _ROWS_PER_TILE // _ZROWS)
    def _z(k):
        pltpu.sync_copy(zbuf, acc.at[pl.ds(tile * _ROWS_PER_TILE + k * _ZROWS, _ZROWS)])


def _sc_deg(dst2d, ew2d, nb):
    nj = _DEG_BLK // 128

    @functools.partial(
        pl.kernel,
        out_type=jax.ShapeDtypeStruct((_NC, _NPAD), jnp.float32),
        mesh=_mesh(),
        compiler_params=pltpu.CompilerParams(use_tc_tiling_on_sc=False),
        scratch_types=[
            pltpu.VMEM_SHARED((_NPAD,), jnp.float32),
            pltpu.VMEM((nj, 128), jnp.int32), pltpu.VMEM((nj, 128), jnp.int32),
            pltpu.VMEM((nj, 128), jnp.float32), pltpu.VMEM((nj, 128), jnp.float32),
            pltpu.VMEM((_NPAD // _NS,), jnp.float32),
            pltpu.SemaphoreType.DMA, pltpu.SemaphoreType.DMA,
        ],
    )
    def k(dst_hbm, ew_hbm, out_hbm, acc, dst0, dst1, ew0, ew1, zbuf,
          isem0, isem1):
        dstb, ewb, isem = (dst0, dst1), (ew0, ew1), (isem0, isem1)
        c = lax.axis_index("c")
        s = lax.axis_index("s")
        w = s * _NC + c
        npt = _NPAD // _NS
        base = w * nb

        @pl.loop(0, npt // 16)
        def _fill(i):
            zbuf[pl.ds(i * 16, 16)] = jnp.zeros((16,), jnp.float32)

        pltpu.sync_copy(zbuf, acc.at[pl.ds(s * npt, npt)])

        plsc.subcore_barrier()

        def stage(b, k_):
            row0 = (base + b) * nj
            pltpu.async_copy(dst_hbm.at[pl.ds(row0, nj)], dstb[k_], isem[k_])
            pltpu.async_copy(ew_hbm.at[pl.ds(row0, nj)], ewb[k_], isem[k_])

        def wait_stage(k_):
            pltpu.make_async_copy(dst_hbm.at[pl.ds(0, nj)], dstb[k_], isem[k_]).wait()
            pltpu.make_async_copy(ew_hbm.at[pl.ds(0, nj)], ewb[k_], isem[k_]).wait()

        stage(0, 0)
        stage(1, 1)

        @pl.loop(0, nb // 2)
        def _blk(t):
            for k_ in (0, 1):
                b = 2 * t + k_
                wait_stage(k_)
                for j in range(nj):
                    pltpu.sync_copy(ewb[k_].at[j], acc.at[dstb[k_].at[j]], add=True)
                stage(jnp.minimum(b + 2, nb - 1), k_)

        wait_stage(0)
        wait_stage(1)

        plsc.subcore_barrier()
        pltpu.sync_copy(acc.at[pl.ds(s * npt, npt)], out_hbm.at[c, pl.ds(s * npt, npt)])

    return k(dst2d, ew2d)


def _sc_edge_pass(h, src2d, dst2d, ew2d, nb, blk, ncols):
    nj = blk // 128
    weighted = ew2d is not None
    scratch = [
        pltpu.VMEM_SHARED((_NPAD, ncols), jnp.float32),
        pltpu.VMEM((nj, 128), jnp.int32), pltpu.VMEM((nj, 128), jnp.int32),
        pltpu.VMEM((nj, 128), jnp.int32), pltpu.VMEM((nj, 128), jnp.int32),
        pltpu.VMEM((blk, ncols), jnp.float32),
        pltpu.VMEM((blk, ncols), jnp.float32),
        pltpu.VMEM((_ZROWS, ncols), jnp.float32),
        pltpu.SemaphoreType.DMA, pltpu.SemaphoreType.DMA,
        pltpu.SemaphoreType.DMA, pltpu.SemaphoreType.DMA,
    ]
    if weighted:
        scratch += [pltpu.VMEM((nj, 128), jnp.float32),
                    pltpu.VMEM((nj, 128), jnp.float32)]

    def body(*args):
        if weighted:
            (h_hbm, src_hbm, dst_hbm, ew_hbm, out_hbm, acc,
             src0, src1, dst0, dst1, rows0, rows1, zbuf,
             gsem0, gsem1, isem0, isem1, ew0, ew1) = args
            ewb = (ew0, ew1)
        else:
            (h_hbm, src_hbm, dst_hbm, out_hbm, acc,
             src0, src1, dst0, dst1, rows0, rows1, zbuf,
             gsem0, gsem1, isem0, isem1) = args
            ewb = (None, None)
        srcb, dstb, rowsb = (src0, src1), (dst0, dst1), (rows0, rows1)
        gsem, isem = (gsem0, gsem1), (isem0, isem1)
        c = lax.axis_index("c")
        s = lax.axis_index("s")
        w = s * _NC + c
        base = w * nb

        _zero_acc2d(zbuf, acc, s, ncols)
        plsc.subcore_barrier()

        def stage(b, k, sem):
            row0 = (base + b) * nj
            if sem is None:
                pltpu.sync_copy(src_hbm.at[pl.ds(row0, nj)], srcb[k])
                pltpu.sync_copy(dst_hbm.at[pl.ds(row0, nj)], dstb[k])
                if weighted:
                    pltpu.sync_copy(ew_hbm.at[pl.ds(row0, nj)], ewb[k])
            else:
                pltpu.async_copy(src_hbm.at[pl.ds(row0, nj)], srcb[k], sem)
                pltpu.async_copy(dst_hbm.at[pl.ds(row0, nj)], dstb[k], sem)
                if weighted:
                    pltpu.async_copy(ew_hbm.at[pl.ds(row0, nj)], ewb[k], sem)

        def wait_stage(k):
            pltpu.make_async_copy(src_hbm.at[pl.ds(0, nj)], srcb[k], isem[k]).wait()
            pltpu.make_async_copy(dst_hbm.at[pl.ds(0, nj)], dstb[k], isem[k]).wait()
            if weighted:
                pltpu.make_async_copy(ew_hbm.at[pl.ds(0, nj)], ewb[k], isem[k]).wait()

        def start_gathers(k):
            for j in range(nj):
                pltpu.async_copy(h_hbm.at[srcb[k].at[j]],
                                 rowsb[k].at[pl.ds(j * 128, 128)], gsem[k])

        def wait_gathers(k):
            for j in range(nj):
                pltpu.make_async_copy(
                    h_hbm.at[srcb[k].at[j]],
                    rowsb[k].at[pl.ds(j * 128, 128)], gsem[k]).wait()

        def scale(k):
            rows, ew_v = rowsb[k], ewb[k]

            @pl.loop(0, blk // 16)
            def _grp(g):
                r = g // 8
                col = (g % 8) * 16
                ewv = ew_v[r, pl.ds(col, 16)]
                for j in range(16):
                    e = g * 16 + j
                    sc = ewv[j]
                    for cc in range(ncols // 16):
                        rows[e, pl.ds(cc * 16, 16)] = (
                            rows[e, pl.ds(cc * 16, 16)] * sc)

        def scatter(k):
            for j in range(nj):
                pltpu.sync_copy(rowsb[k].at[pl.ds(j * 128, 128)],
                                acc.at[dstb[k].at[j]], add=True)

        stage(0, 0, None)
        start_gathers(0)
        stage(1, 1, isem[1])

        @pl.loop(0, nb // 2)
        def _outer(t):
            for k in (0, 1):
                b = 2 * t + k
                wait_gathers(k)
                wait_stage(k ^ 1)
                start_gathers(k ^ 1)
                if weighted:
                    scale(k)
                scatter(k)
                stage(jnp.minimum(b + 2, nb - 1), k, isem[k])

        wait_gathers(0)
        wait_stage(1)

        plsc.subcore_barrier()
        pltpu.sync_copy(acc.at[pl.ds(s * _ROWS_PER_TILE, _ROWS_PER_TILE)],
                        out_hbm.at[c, pl.ds(s * _ROWS_PER_TILE, _ROWS_PER_TILE)])

    kern = functools.partial(
        pl.kernel,
        out_type=jax.ShapeDtypeStruct((_NC, _NPAD, ncols), jnp.float32),
        mesh=_mesh(),
        compiler_params=pltpu.CompilerParams(use_tc_tiling_on_sc=False),
        scratch_types=scratch,
    )(body)
    if weighted:
        return kern(h, src2d, dst2d, ew2d)
    return kern(h, src2d, dst2d)


def _elu(a):
    return jnp.where(a > 0, a, jnp.exp(jnp.minimum(a, 0.0)) - 1.0)


_TCBLK = 8192


def _tc_call(body, n, outd, *args):
    grid = ((n + _TCBLK - 1) // _TCBLK,)
    arrs, specs = zip(*args)
    return pl.pallas_call(
        body,
        grid=grid,
        in_specs=list(specs),
        out_specs=pl.BlockSpec((_TCBLK, outd), lambda i: (i, 0)),
        out_shape=jax.ShapeDtypeStruct((n, outd), jnp.float32),
    )(*arrs)


def _row_spec(d):
    return pl.BlockSpec((_TCBLK, d), lambda i: (i, 0))


def _full_spec(s0, s1):
    return pl.BlockSpec((s0, s1), lambda i: (0, 0))


def _part_spec(d, which):
    return pl.BlockSpec((1, _TCBLK, d), lambda i, _w=which: (_w, i, 0))


def kernel(x_ho, edge_index, edge_weight, x_fo, edge_index_fo, edge_weight_fo,
           edge_index_hon_to_fon, num_ho_nodes, num_fo_nodes,
           W220, b220, W221, b221, W110, b110, W111, b111,
           Wl1, bl1, Wl2, bl2, Wmlp, bmlp):
    n_ho = x_ho.shape[0]
    n_fo = x_fo.shape[0]

    src2d, dst2d, ew2d = _pad_edges(
        edge_index[0], edge_index[1], edge_weight, _EP_HO, 0)
    bsrc2d, bdst2d = _pad_edges(
        edge_index_hon_to_fon[0], edge_index_hon_to_fon[1], None, _EP_BIP, n_ho)

    degp = _sc_deg(dst2d, ew2d, _EP_HO // (_NW * _DEG_BLK)).reshape(_NC, _NPAD, 1)

    def b_body(d0, d1, x_r, w_r, g_r, dis_r):
        deg = d0[0] + d1[0] + 1.0
        dis = lax.rsqrt(deg)
        h = jnp.dot(x_r[...], w_r[...], preferred_element_type=jnp.float32)
        g_r[...] = dis * h
        dis_r[...] = dis

    grid = ((n_ho + _TCBLK - 1) // _TCBLK,)
    g1, dis = pl.pallas_call(
        b_body,
        grid=grid,
        in_specs=[_part_spec(1, 0), _part_spec(1, 1),
                  _row_spec(x_ho.shape[1]), _full_spec(*W220.shape)],
        out_specs=[_row_spec(W220.shape[1]), _row_spec(1)],
        out_shape=[jax.ShapeDtypeStruct((n_ho, W220.shape[1]), jnp.float32),
                   jax.ShapeDtypeStruct((n_ho, 1), jnp.float32)],
    )(degp, degp, x_ho, W220)

    a1 = _sc_edge_pass(g1, src2d, dst2d, ew2d, _EP_HO // (_NW * 1024), 1024, W220.shape[1])

    def d_body(a0, a1_, g_r, dis_r, b_r, w_r, o_r):
        dis = dis_r[...]
        x = _elu(dis * (a0[0] + a1_[0] + g_r[...]) + b_r[...])
        o_r[...] = dis * jnp.dot(x, w_r[...], preferred_element_type=jnp.float32)

    g2 = _tc_call(d_body, n_ho, W221.shape[1],
                  (a1, _part_spec(W220.shape[1], 0)),
                  (a1, _part_spec(W220.shape[1], 1)),
                  (g1, _row_spec(W220.shape[1])),
                  (dis, _row_spec(1)),
                  (b220.reshape(1, -1), _full_spec(1, b220.shape[0])),
                  (W221, _full_spec(*W221.shape)))

    a2 = _sc_edge_pass(g2, src2d, dst2d, ew2d, _EP_HO // (_NW * 256), 256, W221.shape[1])

    def f_body(a0, a1_, g_r, dis_r, b_r, w_r, bl_r, o_r):
        dis = dis_r[...]
        x2 = _elu(dis * (a0[0] + a1_[0] + g_r[...]) + b_r[...])
        o_r[...] = jnp.dot(x2, w_r[...], preferred_element_type=jnp.float32) + bl_r[...]

    h_src = _tc_call(f_body, n_ho, Wl1.shape[1],
                     (a2, _part_spec(W221.shape[1], 0)),
                     (a2, _part_spec(W221.shape[1], 1)),
                     (g2, _row_spec(W221.shape[1])),
                     (dis, _row_spec(1)),
                     (b221.reshape(1, -1), _full_spec(1, b221.shape[0])),
                     (Wl1, _full_spec(*Wl1.shape)),
                     (bl1.reshape(1, -1), _full_spec(1, bl1.shape[0])))

    a3 = _sc_edge_pass(h_src, bsrc2d, bdst2d, None, _EP_BIP // (_NW * 256), 256, Wl1.shape[1])

    def h_body(a0, a1_, w_r, b_r, o_r):
        a = _elu(a0[0] + a1_[0])
        o_r[...] = jnp.dot(a, w_r[...], preferred_element_type=jnp.float32) + b_r[...]

    out_top = _tc_call(h_body, n_ho, Wmlp.shape[1],
                       (a3, _part_spec(Wl1.shape[1], 0)),
                       (a3, _part_spec(Wl1.shape[1], 1)),
                       (Wmlp, _full_spec(*Wmlp.shape)),
                       (bmlp.reshape(1, -1), _full_spec(1, bmlp.shape[0])))

    out_bot = jnp.broadcast_to(bmlp, (n_fo - n_ho, bmlp.shape[0]))
    return jnp.concatenate([out_top, out_bot], axis=0)
